# baseline (speedup 1.0000x reference)
import functools

import jax
import jax.numpy as jnp
from jax import lax
from jax.experimental import pallas as pl


def _mm_kernel(x_ref, w_ref, b_ref, o_ref, *, act):
    acc = jnp.dot(x_ref[...], w_ref[...], preferred_element_type=jnp.float32)
    acc = acc + b_ref[...]
    if act == "leaky":
        acc = jnp.where(acc >= 0, acc, 0.01 * acc)
    elif act == "tanh":
        acc = jnp.tanh(acc)
    o_ref[...] = acc


def _mm(x2d, w2d, b, act, bm):
    M, K = x2d.shape
    N = w2d.shape[1]
    return pl.pallas_call(
        functools.partial(_mm_kernel, act=act),
        grid=(M // bm,),
        in_specs=[
            pl.BlockSpec((bm, K), lambda i: (i, 0)),
            pl.BlockSpec((K, N), lambda i: (0, 0)),
            pl.BlockSpec((1, N), lambda i: (0, 0)),
        ],
        out_specs=pl.BlockSpec((bm, N), lambda i: (i, 0)),
        out_shape=jax.ShapeDtypeStruct((M, N), jnp.float32),
    )(x2d, w2d, b.reshape(1, N))


def _mm_kernel_b(x_ref, w_ref, b_ref, o_ref, *, act):
    acc = jnp.dot(x_ref[0], w_ref[0], preferred_element_type=jnp.float32)
    acc = acc + b_ref[...]
    if act == "leaky":
        acc = jnp.where(acc >= 0, acc, 0.01 * acc)
    elif act == "tanh":
        acc = jnp.tanh(acc)
    o_ref[0] = acc


def _mm_batched(x3d, w3d, b, act, bm):
    P, M, K = x3d.shape
    N = w3d.shape[2]
    return pl.pallas_call(
        functools.partial(_mm_kernel_b, act=act),
        grid=(P, M // bm),
        in_specs=[
            pl.BlockSpec((1, bm, K), lambda p, i: (p, i, 0)),
            pl.BlockSpec((1, K, N), lambda p, i: (p, 0, 0)),
            pl.BlockSpec((1, N), lambda p, i: (0, 0)),
        ],
        out_specs=pl.BlockSpec((1, bm, N), lambda p, i: (p, i, 0)),
        out_shape=jax.ShapeDtypeStruct((P, M, N), jnp.float32),
    )(x3d, w3d, b.reshape(1, N))


def _vq_kernel(f_ref, e_ref, q_ref, loss_ref):
    i = pl.program_id(0)
    f = f_ref[...]
    e = e_ref[...]
    dist = (jnp.sum(f ** 2, axis=1, keepdims=True) + jnp.sum(e ** 2, axis=1)
            - 2.0 * jnp.dot(f, e.T, preferred_element_type=jnp.float32))
    m = jnp.min(dist, axis=1)
    iota = lax.broadcasted_iota(jnp.int32, dist.shape, 1)
    masked = jnp.where(dist == m[:, None], iota, dist.shape[1])
    idx = jnp.min(masked, axis=1)
    oh = (iota == idx[:, None]).astype(jnp.float32)
    q = jnp.dot(oh, e, preferred_element_type=jnp.float32)
    q_ref[...] = q
    r = q - f
    part = jnp.sum(r * r)

    @pl.when(i == 0)
    def _init():
        loss_ref[...] = jnp.zeros_like(loss_ref)

    loss_ref[...] += part


def _vq(flat, embedding, bm):
    M, D = flat.shape
    K = embedding.shape[0]
    q, loss = pl.pallas_call(
        _vq_kernel,
        grid=(M // bm,),
        in_specs=[
            pl.BlockSpec((bm, D), lambda i: (i, 0)),
            pl.BlockSpec((K, D), lambda i: (0, 0)),
        ],
        out_specs=[
            pl.BlockSpec((bm, D), lambda i: (i, 0)),
            pl.BlockSpec((1, 128), lambda i: (0, 0)),
        ],
        out_shape=[
            jax.ShapeDtypeStruct((M, D), jnp.float32),
            jax.ShapeDtypeStruct((1, 128), jnp.float32),
        ],
    )(flat, embedding)
    return q, loss


def _conv_patches(x, kh, kw):
    B, H, W, C = x.shape
    Ho, Wo = H // 2, W // 2
    xp = jnp.pad(x, ((0, 0), (1, 1), (1, 1), (0, 0)))
    cols = []
    for ky in range(kh):
        for kx in range(kw):
            cols.append(xp[:, ky::2, kx::2, :][:, :Ho, :Wo, :])
    pat = jnp.stack(cols, axis=3)
    return pat.reshape(B * Ho * Wo, kh * kw * C)


def _convT_patches(x):
    B, H, W, C = x.shape
    xp = jnp.pad(x, ((0, 0), (1, 1), (1, 1), (0, 0)))
    out = []
    for py in range(2):
        for px in range(2):
            cols = []
            for dy in range(2):
                for dx in range(2):
                    cols.append(xp[:, py + dy: py + dy + H, px + dx: px + dx + W, :])
            pat = jnp.stack(cols, axis=3)
            out.append(pat.reshape(B * H * W, 4 * C))
    return jnp.stack(out, axis=0)


def _convT_weights(w):
    ws = []
    for py in range(2):
        for px in range(2):
            sub = w[py::2, px::2]
            ws.append(sub.reshape(-1, w.shape[3]))
    return jnp.stack(ws, axis=0)


def _interleave(o4, B, H, W, N):
    o = o4.reshape(2, 2, B, H, W, N)
    o = jnp.transpose(o, (2, 3, 0, 4, 1, 5))
    return o.reshape(B, 2 * H, 2 * W, N)


def kernel(x, w1, b1, w2, b2, w3, b3, embedding, dw1, db1, dw2, db2):
    beta = 0.25
    B, H, W, Cin = x.shape
    h1 = w1.shape[3]
    h2 = w2.shape[3]
    D = w3.shape[3]
    K = embedding.shape[0]

    p1 = _conv_patches(x, 4, 4)
    e1 = _mm(p1, w1.reshape(-1, h1), b1, "leaky", 1024)
    e1 = e1.reshape(B, H // 2, W // 2, h1)

    p2 = _conv_patches(e1, 4, 4)
    e2 = _mm(p2, w2.reshape(-1, h2), b2, "leaky", 512)

    enc = _mm(e2, w3.reshape(-1, D), b3, "none", 3584)

    Hq, Wq = H // 4, W // 4
    lat = enc.reshape(B, Hq, Wq, D).transpose(0, 2, 3, 1)
    flat = lat.reshape(-1, D)
    qf, loss_sum = _vq(flat, embedding, 512)
    vq_loss = (1.0 + beta) * loss_sum[0, 0] / (flat.shape[0] * D)
    q = qf.reshape(B, Wq, D, Hq).transpose(0, 3, 1, 2)

    pt1 = _convT_patches(q)
    wt1 = _convT_weights(dw1)
    d1 = _mm_batched(pt1, wt1, db1, "leaky", 512)
    d1 = _interleave(d1, B, Hq, Wq, h2)

    pt2 = _convT_patches(d1)
    wt2 = _convT_weights(dw2)
    Npad = 128
    wt2p = jnp.pad(wt2, ((0, 0), (0, 0), (0, Npad - wt2.shape[2])))
    db2p = jnp.pad(db2, (0, Npad - db2.shape[0]))
    d2 = _mm_batched(pt2, wt2p, db2p, "tanh", 512)
    recon = _interleave(d2, B, H // 2, W // 2, Npad)[..., :Cin]

    return recon, vq_loss
